# baseline (speedup 1.0000x reference)
import dataclasses
import functools

import jax
import jax.numpy as jnp
from jax import lax
from jax.experimental import pallas as pl
from jax.experimental.pallas import tpu as pltpu
from jax.experimental.pallas import tpu_sc as plsc

NUM_CLASSES = 21
TOP_K = 1000
NMS_THRESH = 0.45
CONF_THRESH = 0.01
OBJ_THRESH = 0.01
V0, V1 = 0.1, 0.2


def _decode(loc, priors):
    cxcy = priors[:, :2] + loc[:, :2] * V0 * priors[:, 2:]
    wh = priors[:, 2:] * jnp.exp(loc[:, 2:] * V1)
    return jnp.concatenate([cxcy, wh], axis=1)


def _center_to_corner(b):
    return jnp.concatenate([b[:, :2] - b[:, 2:] / 2.0, b[:, :2] + b[:, 2:] / 2.0], axis=1)


_NMS_BLOCK = 128


def _nms_body(x1_ref, y1_ref, x2_ref, y2_ref, s_ref, keep_ref):
    K, T = x1_ref.shape
    keep_ref[...] = (s_ref[...] > 0.0).astype(jnp.float32)

    for base in range(0, K, _NMS_BLOCK):
        rows = K - base
        rowid = jax.lax.broadcasted_iota(jnp.int32, (rows, T), 0) + base

        def body(i, _, base=base, rows=rows, rowid=rowid):
            x1 = x1_ref[base:, :]
            y1 = y1_ref[base:, :]
            x2 = x2_ref[base:, :]
            y2 = y2_ref[base:, :]
            x1i = x1_ref[pl.ds(i, 1), :]
            y1i = y1_ref[pl.ds(i, 1), :]
            x2i = x2_ref[pl.ds(i, 1), :]
            y2i = y2_ref[pl.ds(i, 1), :]
            ai = jnp.maximum(x2i - x1i, 0.0) * jnp.maximum(y2i - y1i, 0.0)
            area = jnp.maximum(x2 - x1, 0.0) * jnp.maximum(y2 - y1, 0.0)
            ki = keep_ref[pl.ds(i, 1), :]
            xx1 = jnp.maximum(x1, x1i)
            yy1 = jnp.maximum(y1, y1i)
            xx2 = jnp.minimum(x2, x2i)
            yy2 = jnp.minimum(y2, y2i)
            inter = jnp.maximum(xx2 - xx1, 0.0) * jnp.maximum(yy2 - yy1, 0.0)
            union = area + ai - inter
            iou = inter / jnp.maximum(union, 1e-9)
            sup = (iou > NMS_THRESH) & (ki > 0.0) & (rowid > i)
            keep_ref[base:, :] = jnp.where(sup, 0.0, keep_ref[base:, :])
            return 0

        jax.lax.fori_loop(base, min(base + _NMS_BLOCK, K), body, 0)


_L = 16
_PPAD = 5008
_NCH = _PPAD // _L
_KPAD = 1008
_KCH = _KPAD // _L
_SORT_BITS = 26


def _sc_compiler_params():
    cp = pltpu.CompilerParams()
    if "needs_layout_passes" in pltpu.CompilerParams.__dataclass_fields__:
        cp = dataclasses.replace(cp, needs_layout_passes=False)
    if "use_tc_tiling_on_sc" in pltpu.CompilerParams.__dataclass_fields__:
        cp = dataclasses.replace(cp, use_tc_tiling_on_sc=False)
    return cp


def _sc_sort_gather(conf_pad, boxes_flat, T):
    mesh = plsc.VectorSubcoreMesh(core_axis_name="c", subcore_axis_name="s")
    n_workers = 32
    n_slots = (T + n_workers - 1) // n_workers

    @functools.partial(
        pl.kernel,
        out_type=[
            jax.ShapeDtypeStruct((T, _KPAD), jnp.float32),
            jax.ShapeDtypeStruct((T, 4, _KPAD), jnp.float32),
        ],
        mesh=mesh,
        scratch_types=[
            pltpu.VMEM((_PPAD,), jnp.float32),
            pltpu.VMEM((_PPAD + _L,), jnp.int32),
            pltpu.VMEM((_PPAD + _L,), jnp.int32),
            pltpu.VMEM((_PPAD + _L,), jnp.int32),
            pltpu.VMEM((_PPAD + _L,), jnp.int32),
            pltpu.VMEM((4, _PPAD), jnp.float32),
            pltpu.VMEM((4, _KPAD), jnp.float32),
            pltpu.SemaphoreType.DMA,
        ],
        compiler_params=_sc_compiler_params(),
    )
    def sc_kernel(conf_hbm, boxes_hbm, vals_hbm, cand_hbm,
                  scores_v, ka, kb, ia, ib, boxes_v, cand_v, sem):
        wid = lax.axis_index("s") * 2 + lax.axis_index("c")

        def do_task(t):
            pltpu.async_copy(conf_hbm.at[t], scores_v, sem).wait()
            b = t // NUM_CLASSES
            pltpu.async_copy(boxes_hbm.at[b], boxes_v, sem).wait()

            for p in range(_SORT_BITS):
                if p == 0:
                    src_k, src_i, dst_k, dst_i = scores_v, None, kb, ib
                elif p % 2 == 1:
                    src_k, src_i, dst_k, dst_i = kb, ib, ka, ia
                else:
                    src_k, src_i, dst_k, dst_i = ka, ia, kb, ib

                def load_kv(j, _p=p, _sk=src_k, _si=src_i):
                    if _p == 0:
                        k = plsc.bitcast(_sk[pl.ds(j * _L, _L)], jnp.int32)
                        iv = lax.iota(jnp.int32, _L) + j * _L
                    else:
                        k = _sk[pl.ds(j * _L, _L)]
                        iv = _si[pl.ds(j * _L, _L)]
                    return k, iv

                def count_body(j, acc, _load=load_kv, _p=p):
                    k, _ = _load(j)
                    bit = (lax.shift_right_logical(k, _p) & 1) == 1
                    return acc + jnp.sum(bit.astype(jnp.int32))

                n1 = lax.fori_loop(0, _NCH, count_body, jnp.int32(0))

                def part_body(j, offs, _load=load_kv, _p=p,
                              _dk=dst_k, _di=dst_i):
                    o1, o0 = offs
                    k, iv = _load(j)
                    m1 = (lax.shift_right_logical(k, _p) & 1) == 1
                    c1 = jnp.sum(m1.astype(jnp.int32))
                    plsc.store_compressed(_dk.at[pl.ds(o1, _L)], k, mask=m1)
                    plsc.store_compressed(_di.at[pl.ds(o1, _L)], iv, mask=m1)
                    m0 = ~m1
                    plsc.store_compressed(_dk.at[pl.ds(o0, _L)], k, mask=m0)
                    plsc.store_compressed(_di.at[pl.ds(o0, _L)], iv, mask=m0)
                    return (o1 + c1, o0 + (_L - c1))

                lax.fori_loop(0, _NCH, part_body, (jnp.int32(0), n1))

            @pl.loop(0, _KCH)
            def _(j):
                sl = pl.ds(j * _L, _L)
                scores_v[sl] = plsc.bitcast(ka[sl], jnp.float32)
                idxv = ia[sl]
                for r in range(4):
                    cand_v[r, sl] = plsc.load_gather(boxes_v.at[r], [idxv])

            pltpu.async_copy(scores_v.at[pl.ds(0, _KPAD)], vals_hbm.at[t], sem).wait()
            pltpu.async_copy(cand_v, cand_hbm.at[t], sem).wait()

        for slot in range(n_slots):
            t = slot * n_workers + wid
            if (slot + 1) * n_workers <= T:
                do_task(t)
            else:
                @pl.when(t < T)
                def _():
                    do_task(t)

    return sc_kernel(conf_pad, boxes_flat)


def _sc_compact(keep_t, vals_t, cand_t, T):
    mesh = plsc.VectorSubcoreMesh(core_axis_name="c", subcore_axis_name="s")
    n_workers = 32
    n_slots = (T + n_workers - 1) // n_workers

    @functools.partial(
        pl.kernel,
        out_type=jax.ShapeDtypeStruct((T, 5 * _KPAD), jnp.float32),
        mesh=mesh,
        scratch_types=[
            pltpu.VMEM((_KPAD,), jnp.float32),
            pltpu.VMEM((_KPAD,), jnp.float32),
            pltpu.VMEM((4, _KPAD), jnp.float32),
            pltpu.VMEM((5 * _KPAD + _L,), jnp.float32),
            pltpu.SemaphoreType.DMA,
        ],
        compiler_params=_sc_compiler_params(),
    )
    def sc_kernel(keep_hbm, vals_hbm, cand_hbm, out_hbm, kv, vv, cv, ov, sem):
        wid = lax.axis_index("s") * 2 + lax.axis_index("c")
        zeros = jnp.zeros((_L,), jnp.float32)

        def do_task(t):
            pltpu.async_copy(keep_hbm.at[t], kv, sem).wait()
            pltpu.async_copy(vals_hbm.at[t], vv, sem).wait()
            pltpu.async_copy(cand_hbm.at[t], cv, sem).wait()

            @pl.loop(0, (5 * _KPAD + _L) // _L)
            def _(j):
                ov[pl.ds(j * _L, _L)] = zeros

            def body(j, off):
                sl = pl.ds(j * _L, _L)
                m = kv[sl] > 0.0
                plsc.store_compressed(ov.at[pl.ds(off, _L)], vv[sl], mask=m)
                for r in range(4):
                    plsc.store_compressed(
                        ov.at[pl.ds((r + 1) * _KPAD + off, _L)], cv[r, sl], mask=m)
                return off + jnp.sum(m.astype(jnp.int32))

            lax.fori_loop(0, _KCH, body, jnp.int32(0))
            pltpu.async_copy(ov.at[pl.ds(0, 5 * _KPAD)], out_hbm.at[t], sem).wait()

        for slot in range(n_slots):
            t = slot * n_workers + wid
            if (slot + 1) * n_workers <= T:
                do_task(t)
            else:
                @pl.when(t < T)
                def _():
                    do_task(t)

    return sc_kernel(keep_t, vals_t, cand_t)


def _nms_keep_pallas(x1, y1, x2, y2, s):
    return pl.pallas_call(
        _nms_body,
        out_shape=jax.ShapeDtypeStruct(x1.shape, jnp.float32),
    )(x1, y1, x2, y2, s)


def kernel(arm_loc_data, arm_conf_data, odm_loc_data, odm_conf_data, prior_data):
    B, P, C = odm_conf_data.shape
    arm_conf = jax.nn.softmax(arm_conf_data, axis=2)
    odm_conf = jax.nn.softmax(odm_conf_data, axis=2)
    obj = arm_conf[:, :, 1]
    odm_conf = jnp.where((obj <= OBJ_THRESH)[:, :, None], 0.0, odm_conf)
    conf_preds = jnp.transpose(odm_conf, (0, 2, 1))

    def decode_image(arm_loc_i, odm_loc_i):
        dec1 = _decode(arm_loc_i, prior_data)
        dec2 = _decode(odm_loc_i, dec1)
        return _center_to_corner(dec2)

    boxes = jax.vmap(decode_image)(arm_loc_data, odm_loc_data)

    sm = jnp.where(conf_preds > CONF_THRESH, conf_preds, 0.0)

    T = B * C
    conf_pad = jnp.pad(sm, ((0, 0), (0, 0), (0, _PPAD - P))).reshape(T, _PPAD)
    boxes_t = jnp.pad(jnp.transpose(boxes, (0, 2, 1)),
                      ((0, 0), (0, 0), (0, _PPAD - P)))
    vals_t, cand_t = _sc_sort_gather(conf_pad, boxes_t, T)

    x1 = cand_t[:, 0, :TOP_K].T
    y1 = cand_t[:, 1, :TOP_K].T
    x2 = cand_t[:, 2, :TOP_K].T
    y2 = cand_t[:, 3, :TOP_K].T
    sv = vals_t[:, :TOP_K].T

    keep_kt = _nms_keep_pallas(x1, y1, x2, y2, sv)
    keep_t = jnp.pad(keep_kt.T, ((0, 0), (0, _KPAD - TOP_K)))

    out5 = _sc_compact(keep_t, vals_t, cand_t, T).reshape(T, 5, _KPAD)
    out = jnp.transpose(out5[:, :, :TOP_K], (0, 2, 1)).reshape(B, C, TOP_K, 5)
    return out.at[:, 0].set(0.0)


# speedup vs baseline: 1.1352x; 1.1352x over previous
import dataclasses
import functools

import jax
import jax.numpy as jnp
from jax import lax
from jax.experimental import pallas as pl
from jax.experimental.pallas import tpu as pltpu
from jax.experimental.pallas import tpu_sc as plsc

NUM_CLASSES = 21
TOP_K = 1000
NMS_THRESH = 0.45
CONF_THRESH = 0.01
OBJ_THRESH = 0.01
V0, V1 = 0.1, 0.2


def _decode(loc, priors):
    cxcy = priors[:, :2] + loc[:, :2] * V0 * priors[:, 2:]
    wh = priors[:, 2:] * jnp.exp(loc[:, 2:] * V1)
    return jnp.concatenate([cxcy, wh], axis=1)


def _center_to_corner(b):
    return jnp.concatenate([b[:, :2] - b[:, 2:] / 2.0, b[:, :2] + b[:, 2:] / 2.0], axis=1)


_NMS_BLOCK = 128


def _nms_body(x1_ref, y1_ref, x2_ref, y2_ref, s_ref, keep_ref):
    K, T = x1_ref.shape
    keep_ref[...] = (s_ref[...] > 0.0).astype(jnp.float32)

    for base in range(0, K, _NMS_BLOCK):
        rows = K - base
        rowid = jax.lax.broadcasted_iota(jnp.int32, (rows, T), 0) + base

        def body(i, _, base=base, rows=rows, rowid=rowid):
            x1 = x1_ref[base:, :]
            y1 = y1_ref[base:, :]
            x2 = x2_ref[base:, :]
            y2 = y2_ref[base:, :]
            x1i = x1_ref[pl.ds(i, 1), :]
            y1i = y1_ref[pl.ds(i, 1), :]
            x2i = x2_ref[pl.ds(i, 1), :]
            y2i = y2_ref[pl.ds(i, 1), :]
            ai = jnp.maximum(x2i - x1i, 0.0) * jnp.maximum(y2i - y1i, 0.0)
            area = jnp.maximum(x2 - x1, 0.0) * jnp.maximum(y2 - y1, 0.0)
            ki = keep_ref[pl.ds(i, 1), :]
            xx1 = jnp.maximum(x1, x1i)
            yy1 = jnp.maximum(y1, y1i)
            xx2 = jnp.minimum(x2, x2i)
            yy2 = jnp.minimum(y2, y2i)
            inter = jnp.maximum(xx2 - xx1, 0.0) * jnp.maximum(yy2 - yy1, 0.0)
            union = area + ai - inter
            iou = inter / jnp.maximum(union, 1e-9)
            sup = (iou > NMS_THRESH) & (ki > 0.0) & (rowid > i)
            keep_ref[base:, :] = jnp.where(sup, 0.0, keep_ref[base:, :])
            return 0

        jax.lax.fori_loop(base, min(base + _NMS_BLOCK, K), body, 0)


_L = 16
_PPAD = 5008
_NCH = _PPAD // _L
_KPAD = 1008
_KCH = _KPAD // _L
_SORT_BITS = 26


def _sc_compiler_params():
    cp = pltpu.CompilerParams()
    if "needs_layout_passes" in pltpu.CompilerParams.__dataclass_fields__:
        cp = dataclasses.replace(cp, needs_layout_passes=False)
    if "use_tc_tiling_on_sc" in pltpu.CompilerParams.__dataclass_fields__:
        cp = dataclasses.replace(cp, use_tc_tiling_on_sc=False)
    return cp


def _sc_sort_gather(conf_pad, boxes_flat, T):
    mesh = plsc.VectorSubcoreMesh(core_axis_name="c", subcore_axis_name="s")
    n_workers = 32
    n_slots = (T + n_workers - 1) // n_workers

    @functools.partial(
        pl.kernel,
        out_type=[
            jax.ShapeDtypeStruct((T, _KPAD), jnp.float32),
            jax.ShapeDtypeStruct((T, 4, _KPAD), jnp.float32),
        ],
        mesh=mesh,
        scratch_types=[
            pltpu.VMEM((_PPAD,), jnp.float32),
            pltpu.VMEM((_PPAD + _L,), jnp.int32),
            pltpu.VMEM((_PPAD + _L,), jnp.int32),
            pltpu.VMEM((_PPAD + _L,), jnp.int32),
            pltpu.VMEM((_PPAD + _L,), jnp.int32),
            pltpu.VMEM((4, _PPAD), jnp.float32),
            pltpu.VMEM((4, _KPAD), jnp.float32),
            pltpu.SemaphoreType.DMA,
        ],
        compiler_params=_sc_compiler_params(),
    )
    def sc_kernel(conf_hbm, boxes_hbm, vals_hbm, cand_hbm,
                  scores_v, ka, kb, ia, ib, boxes_v, cand_v, sem):
        wid = lax.axis_index("s") * 2 + lax.axis_index("c")

        def do_task(t):
            cp_conf = pltpu.async_copy(conf_hbm.at[t], scores_v, sem)
            b = t // NUM_CLASSES
            cp_box = pltpu.async_copy(boxes_hbm.at[b], boxes_v, sem)
            cp_conf.wait()
            cp_box.wait()

            def count_body(j, acc):
                k = plsc.bitcast(scores_v[pl.ds(j * _L, _L)], jnp.int32)
                return acc + jnp.sum((k & 1).astype(jnp.int32))

            n1 = lax.fori_loop(0, _NCH, count_body, jnp.int32(0))

            for p in range(_SORT_BITS):
                if p == 0:
                    src_k, src_i, dst_k, dst_i = scores_v, None, kb, ib
                elif p % 2 == 1:
                    src_k, src_i, dst_k, dst_i = kb, ib, ka, ia
                else:
                    src_k, src_i, dst_k, dst_i = ka, ia, kb, ib

                def load_kv(j, _p=p, _sk=src_k, _si=src_i):
                    if _p == 0:
                        k = plsc.bitcast(_sk[pl.ds(j * _L, _L)], jnp.int32)
                        iv = lax.iota(jnp.int32, _L) + j * _L
                    else:
                        k = _sk[pl.ds(j * _L, _L)]
                        iv = _si[pl.ds(j * _L, _L)]
                    return k, iv

                def part_body(j, offs, _load=load_kv, _p=p,
                              _dk=dst_k, _di=dst_i):
                    o1, o0, nxt = offs
                    k, iv = _load(j)
                    m1 = (lax.shift_right_logical(k, _p) & 1) == 1
                    c1 = jnp.sum(m1.astype(jnp.int32))
                    plsc.store_compressed(_dk.at[pl.ds(o1, _L)], k, mask=m1)
                    plsc.store_compressed(_di.at[pl.ds(o1, _L)], iv, mask=m1)
                    m0 = ~m1
                    plsc.store_compressed(_dk.at[pl.ds(o0, _L)], k, mask=m0)
                    plsc.store_compressed(_di.at[pl.ds(o0, _L)], iv, mask=m0)
                    cn = jnp.sum((lax.shift_right_logical(k, _p + 1) & 1)
                                 .astype(jnp.int32))
                    return (o1 + c1, o0 + (_L - c1), nxt + cn)

                _, _, n1 = lax.fori_loop(0, _NCH, part_body,
                                         (jnp.int32(0), n1, jnp.int32(0)))

            @pl.loop(0, _KCH)
            def _(j):
                sl = pl.ds(j * _L, _L)
                scores_v[sl] = plsc.bitcast(ka[sl], jnp.float32)
                idxv = ia[sl]
                for r in range(4):
                    cand_v[r, sl] = plsc.load_gather(boxes_v.at[r], [idxv])

            pltpu.async_copy(scores_v.at[pl.ds(0, _KPAD)], vals_hbm.at[t], sem).wait()
            pltpu.async_copy(cand_v, cand_hbm.at[t], sem).wait()

        for slot in range(n_slots):
            t = slot * n_workers + wid
            if (slot + 1) * n_workers <= T:
                do_task(t)
            else:
                @pl.when(t < T)
                def _():
                    do_task(t)

    return sc_kernel(conf_pad, boxes_flat)


def _sc_compact(keep_t, vals_t, cand_t, T):
    mesh = plsc.VectorSubcoreMesh(core_axis_name="c", subcore_axis_name="s")
    n_workers = 32
    n_slots = (T + n_workers - 1) // n_workers

    @functools.partial(
        pl.kernel,
        out_type=jax.ShapeDtypeStruct((T, 5 * _KPAD), jnp.float32),
        mesh=mesh,
        scratch_types=[
            pltpu.VMEM((_KPAD,), jnp.float32),
            pltpu.VMEM((_KPAD,), jnp.float32),
            pltpu.VMEM((4, _KPAD), jnp.float32),
            pltpu.VMEM((5 * _KPAD + _L,), jnp.float32),
            pltpu.SemaphoreType.DMA,
        ],
        compiler_params=_sc_compiler_params(),
    )
    def sc_kernel(keep_hbm, vals_hbm, cand_hbm, out_hbm, kv, vv, cv, ov, sem):
        wid = lax.axis_index("s") * 2 + lax.axis_index("c")
        zeros = jnp.zeros((_L,), jnp.float32)

        def do_task(t):
            pltpu.async_copy(keep_hbm.at[t], kv, sem).wait()
            pltpu.async_copy(vals_hbm.at[t], vv, sem).wait()
            pltpu.async_copy(cand_hbm.at[t], cv, sem).wait()

            @pl.loop(0, (5 * _KPAD + _L) // _L)
            def _(j):
                ov[pl.ds(j * _L, _L)] = zeros

            def body(j, off):
                sl = pl.ds(j * _L, _L)
                m = kv[sl] > 0.0
                plsc.store_compressed(ov.at[pl.ds(off, _L)], vv[sl], mask=m)
                for r in range(4):
                    plsc.store_compressed(
                        ov.at[pl.ds((r + 1) * _KPAD + off, _L)], cv[r, sl], mask=m)
                return off + jnp.sum(m.astype(jnp.int32))

            lax.fori_loop(0, _KCH, body, jnp.int32(0))
            pltpu.async_copy(ov.at[pl.ds(0, 5 * _KPAD)], out_hbm.at[t], sem).wait()

        for slot in range(n_slots):
            t = slot * n_workers + wid
            if (slot + 1) * n_workers <= T:
                do_task(t)
            else:
                @pl.when(t < T)
                def _():
                    do_task(t)

    return sc_kernel(keep_t, vals_t, cand_t)


def _nms_keep_pallas(x1, y1, x2, y2, s):
    return pl.pallas_call(
        _nms_body,
        out_shape=jax.ShapeDtypeStruct(x1.shape, jnp.float32),
    )(x1, y1, x2, y2, s)


def kernel(arm_loc_data, arm_conf_data, odm_loc_data, odm_conf_data, prior_data):
    B, P, C = odm_conf_data.shape
    arm_conf = jax.nn.softmax(arm_conf_data, axis=2)
    odm_conf = jax.nn.softmax(odm_conf_data, axis=2)
    obj = arm_conf[:, :, 1]
    odm_conf = jnp.where((obj <= OBJ_THRESH)[:, :, None], 0.0, odm_conf)
    conf_preds = jnp.transpose(odm_conf, (0, 2, 1))

    def decode_image(arm_loc_i, odm_loc_i):
        dec1 = _decode(arm_loc_i, prior_data)
        dec2 = _decode(odm_loc_i, dec1)
        return _center_to_corner(dec2)

    boxes = jax.vmap(decode_image)(arm_loc_data, odm_loc_data)

    sm = jnp.where(conf_preds > CONF_THRESH, conf_preds, 0.0)

    T = B * C
    conf_pad = jnp.pad(sm, ((0, 0), (0, 0), (0, _PPAD - P))).reshape(T, _PPAD)
    boxes_t = jnp.pad(jnp.transpose(boxes, (0, 2, 1)),
                      ((0, 0), (0, 0), (0, _PPAD - P)))
    vals_t, cand_t = _sc_sort_gather(conf_pad, boxes_t, T)

    x1 = cand_t[:, 0, :TOP_K].T
    y1 = cand_t[:, 1, :TOP_K].T
    x2 = cand_t[:, 2, :TOP_K].T
    y2 = cand_t[:, 3, :TOP_K].T
    sv = vals_t[:, :TOP_K].T

    keep_kt = _nms_keep_pallas(x1, y1, x2, y2, sv)
    keep_t = jnp.pad(keep_kt.T, ((0, 0), (0, _KPAD - TOP_K)))

    out5 = _sc_compact(keep_t, vals_t, cand_t, T).reshape(T, 5, _KPAD)
    out = jnp.transpose(out5[:, :, :TOP_K], (0, 2, 1)).reshape(B, C, TOP_K, 5)
    return out.at[:, 0].set(0.0)
